# baseline (speedup 1.0000x reference)
import functools

import jax
import jax.numpy as jnp
from jax import lax
from jax.experimental import pallas as pl
from jax.experimental.pallas import tpu as pltpu
from jax.experimental.pallas import tpu_sc as plsc

_VOCAB = 1000000
_D = 64
_C = 2
_B = 4096
_SEQ = 200
_HALF = _SEQ // 2

_NC = 2
_NS = 16
_NW = _NC * _NS
_BPW = _B // _NW
_LANES = 16


def _pool_body(x_hbm, table_hbm, out_hbm, idx_v, rows_v, pooled_v, sem0, sem1):
    wid = lax.axis_index("s") * _NC + lax.axis_index("c")
    base = wid * _BPW

    pltpu.sync_copy(x_hbm.at[pl.ds(base * 2, _BPW * 2)], idx_v)

    sems = (sem0, sem1)

    def gathers(e, p):
        out = []
        for h in range(2):
            out.append(pltpu.make_async_copy(
                table_hbm.at[idx_v.at[2 * e + h]],
                rows_v.at[pl.ds(p * _SEQ + h * _HALF, _HALF)],
                sems[p]))
        return out

    def fire(e, p):
        for c in gathers(e, p):
            c.start()

    def wait(e, p):
        for c in gathers(e, p):
            c.wait()

    zero = jnp.zeros((_LANES,), jnp.float32)

    def reduce_into(e, p):
        rbase = p * _SEQ

        def rbody(j, carry):
            a0, a1, a2, a3 = carry
            r = rbase + j
            a0 = a0 + rows_v[r, pl.ds(0, 16)]
            a1 = a1 + rows_v[r, pl.ds(16, 16)]
            a2 = a2 + rows_v[r, pl.ds(32, 16)]
            a3 = a3 + rows_v[r, pl.ds(48, 16)]
            return (a0, a1, a2, a3)

        a0, a1, a2, a3 = lax.fori_loop(0, _SEQ, rbody, (zero, zero, zero, zero))
        pooled_v[pl.ds(e * _D + 0, 16)] = a0
        pooled_v[pl.ds(e * _D + 16, 16)] = a1
        pooled_v[pl.ds(e * _D + 32, 16)] = a2
        pooled_v[pl.ds(e * _D + 48, 16)] = a3

    fire(0, 0)

    def outer(i, _):
        for p in range(2):
            e = 2 * i + p

            @pl.when(e + 1 < _BPW)
            def _fire():
                fire(e + 1, 1 - p)

            wait(e, p)
            reduce_into(e, p)
        return 0

    lax.fori_loop(0, _BPW // 2, outer, 0)

    pltpu.sync_copy(pooled_v, out_hbm.at[pl.ds(base * _D, _BPW * _D)])


@functools.partial(
    pl.kernel,
    out_type=jax.ShapeDtypeStruct((_B * _D,), jnp.float32),
    mesh=plsc.VectorSubcoreMesh(core_axis_name="c", subcore_axis_name="s"),
    scratch_types=[
        pltpu.VMEM((2 * _BPW, _HALF), jnp.int32),
        pltpu.VMEM((2 * _SEQ, _D), jnp.float32),
        pltpu.VMEM((_BPW * _D,), jnp.float32),
        pltpu.SemaphoreType.DMA,
        pltpu.SemaphoreType.DMA,
    ],
    compiler_params=pltpu.CompilerParams(use_tc_tiling_on_sc=False),
)
def _sc_pool(x_hbm, table_hbm, out_hbm, idx_v, rows_v, pooled_v, sem0, sem1):
    _pool_body(x_hbm, table_hbm, out_hbm, idx_v, rows_v, pooled_v, sem0, sem1)


def _proj_body(p_ref, w_ref, b_ref, o_ref):
    o_ref[...] = (
        jnp.dot(p_ref[...], w_ref[...], preferred_element_type=jnp.float32)
        + b_ref[...]
    )


_proj = pl.pallas_call(
    _proj_body,
    out_shape=jax.ShapeDtypeStruct((_B, _C), jnp.float32),
)


def kernel(x, table, W, b):
    x2 = x.reshape(_B * 2, _HALF).astype(jnp.int32)
    pooled = _sc_pool(x2, table).reshape(_B, _D)
    w_scaled = (W.T * (1.0 / _SEQ)).astype(jnp.float32)
    return _proj(pooled, w_scaled, b.astype(jnp.float32)[None, :])


# speedup vs baseline: 2.4507x; 2.4507x over previous
import functools

import jax
import jax.numpy as jnp
from jax import lax
from jax.experimental import pallas as pl
from jax.experimental.pallas import tpu as pltpu
from jax.experimental.pallas import tpu_sc as plsc

_VOCAB = 1000000
_D = 64
_C = 2
_B = 4096
_SEQ = 200
_HALF = _SEQ // 2

_NC = 2
_NS = 16
_NW = _NC * _NS
_BPW = _B // _NW
_LANES = 16

_NB = 8192
_GRID = (_VOCAB + _NB - 1) // _NB


def _proj_body(w_ref, tt_ref, o_ref):
    o_ref[...] = jnp.dot(w_ref[...], tt_ref[...],
                         preferred_element_type=jnp.float32)


_proj = pl.pallas_call(
    _proj_body,
    grid=(_GRID,),
    in_specs=[
        pl.BlockSpec((8, _D), lambda i: (0, 0)),
        pl.BlockSpec((_D, _NB), lambda i: (0, i)),
    ],
    out_specs=pl.BlockSpec((8, _NB), lambda i: (0, i)),
    out_shape=jax.ShapeDtypeStruct((8, _VOCAB), jnp.float32),
)

_BUF = 208
_OFF1 = 104


def _pool_body(x_hbm, t0_hbm, t1_hbm, b_hbm, out_hbm,
               idx_v, rows_v, b_v, out_v, sem0, sem1):
    wid = lax.axis_index("s") * _NC + lax.axis_index("c")
    base = wid * _BPW

    pltpu.sync_copy(x_hbm.at[pl.ds(base * 2, _BPW * 2)], idx_v)
    pltpu.sync_copy(b_hbm, b_v)

    planes = (t0_hbm, t1_hbm)
    sems = (sem0, sem1)

    def copies(e, par):
        out = []
        for p in range(2):
            for h in range(2):
                out.append(pltpu.make_async_copy(
                    planes[p].at[idx_v.at[2 * e + h]],
                    rows_v.at[pl.ds(par * 2 * _BUF + p * _BUF + h * _OFF1,
                                    _HALF)],
                    sems[par]))
        return out

    def fire(e, par):
        for c in copies(e, par):
            c.start()

    def wait(e, par):
        for c in copies(e, par):
            c.wait()

    lane = lax.iota(jnp.int32, _LANES)
    m6 = (lane < 4) | (lane >= 8)
    m12 = lane < 12
    zero = jnp.zeros((_LANES,), jnp.float32)

    def reduce_plane(par, p):
        rb = par * 2 * _BUF + p * _BUF
        acc = zero
        for k in range(13):
            v = rows_v[pl.ds(rb + k * 16, 16)]
            if k == 6:
                v = jnp.where(m6, v, 0.0)
            elif k == 12:
                v = jnp.where(m12, v, 0.0)
            acc = acc + v
        return jnp.full((_LANES,), jnp.sum(acc), jnp.float32) + b_v[p, :]

    fire(0, 0)

    def outer(i, _):
        for par in range(2):
            e = 2 * i + par

            @pl.when(e + 1 < _BPW)
            def _fire():
                fire(e + 1, 1 - par)

            wait(e, par)
            out_v[pl.ds(e * 2 * _LANES, _LANES)] = reduce_plane(par, 0)
            out_v[pl.ds(e * 2 * _LANES + _LANES, _LANES)] = reduce_plane(par, 1)
        return 0

    lax.fori_loop(0, _BPW // 2, outer, 0)

    pltpu.sync_copy(out_v, out_hbm.at[pl.ds(base * 2 * _LANES,
                                            _BPW * 2 * _LANES)])


@functools.partial(
    pl.kernel,
    out_type=jax.ShapeDtypeStruct((_B * _C * _LANES,), jnp.float32),
    mesh=plsc.VectorSubcoreMesh(core_axis_name="c", subcore_axis_name="s"),
    scratch_types=[
        pltpu.VMEM((2 * _BPW, _HALF), jnp.int32),
        pltpu.VMEM((4 * _BUF,), jnp.float32),
        pltpu.VMEM((_C, _LANES), jnp.float32),
        pltpu.VMEM((_BPW * _C * _LANES,), jnp.float32),
        pltpu.SemaphoreType.DMA,
        pltpu.SemaphoreType.DMA,
    ],
    compiler_params=pltpu.CompilerParams(use_tc_tiling_on_sc=False,
                                         needs_layout_passes=False),
)
def _sc_pool(x_hbm, t0_hbm, t1_hbm, b_hbm, out_hbm,
             idx_v, rows_v, b_v, out_v, sem0, sem1):
    _pool_body(x_hbm, t0_hbm, t1_hbm, b_hbm, out_hbm,
               idx_v, rows_v, b_v, out_v, sem0, sem1)


def kernel(x, table, W, b):
    w8 = jnp.zeros((8, _D), jnp.float32).at[:_C].set(
        W.astype(jnp.float32) * (1.0 / _SEQ))
    tw8 = _proj(w8, table.T)
    x2 = x.reshape(_B * 2, _HALF).astype(jnp.int32)
    b_bcast = jnp.broadcast_to(b.astype(jnp.float32)[:, None], (_C, _LANES))
    out = _sc_pool(x2, tw8[0], tw8[1], b_bcast)
    return out.reshape(_B, _C, _LANES)[:, :, 0]


# speedup vs baseline: 3.4941x; 1.4258x over previous
import functools

import jax
import jax.numpy as jnp
from jax import lax
from jax.experimental import pallas as pl
from jax.experimental.pallas import tpu as pltpu
from jax.experimental.pallas import tpu_sc as plsc

_VOCAB = 1000000
_D = 64
_C = 2
_B = 4096
_SEQ = 200

_NC = 2
_NS = 16
_NW = _NC * _NS
_BPW = _B // _NW
_LANES = 16
_GPW = _BPW // _LANES

_NB = 8192
_GRID = (_VOCAB + _NB - 1) // _NB


def _proj_body(w_ref, tt_ref, o0_ref, o1_ref):
    res = jnp.dot(w_ref[...], tt_ref[...], preferred_element_type=jnp.float32)
    o0_ref[...] = res[0, :]
    o1_ref[...] = res[1, :]


_proj = pl.pallas_call(
    _proj_body,
    grid=(_GRID,),
    in_specs=[
        pl.BlockSpec((8, _D), lambda i: (0, 0)),
        pl.BlockSpec((_D, _NB), lambda i: (0, i)),
    ],
    out_specs=[
        pl.BlockSpec((_NB,), lambda i: (i,)),
        pl.BlockSpec((_NB,), lambda i: (i,)),
    ],
    out_shape=[
        jax.ShapeDtypeStruct((_VOCAB,), jnp.float32),
        jax.ShapeDtypeStruct((_VOCAB,), jnp.float32),
    ],
)

_WAVE = 20
_NWAVE = _SEQ // _WAVE


def _pool_body(xt_hbm, t0_hbm, t1_hbm, b_hbm, out_hbm,
               idx_v, g0_v, g1_v, b_v, o0_v, o1_v, sem0, sem1):
    wid = lax.axis_index("s") * _NC + lax.axis_index("c")
    base = wid * _BPW

    pltpu.sync_copy(xt_hbm.at[:, pl.ds(base, _BPW)], idx_v)
    pltpu.sync_copy(b_hbm, b_v)

    sems = (sem0, sem1)
    planes = ((t0_hbm, g0_v), (t1_hbm, g1_v))

    def copies(j, sem):
        return [pltpu.make_async_copy(src.at[idx_v.at[j]],
                                      buf.at[pl.ds(j * _BPW, _BPW)], sem)
                for src, buf in planes]

    def fire_wave(w, par):
        def f1(j, _):
            for c in copies(j, sems[par]):
                c.start()
            return 0
        lax.fori_loop(w * _WAVE, (w + 1) * _WAVE, f1, 0)

    def wait_wave(w, par):
        def f1(j, _):
            for c in copies(j, sems[par]):
                c.wait()
            return 0
        lax.fori_loop(w * _WAVE, (w + 1) * _WAVE, f1, 0)

    fire_wave(0, 0)

    def outer(i, _):
        for par in range(2):
            w = 2 * i + par

            @pl.when(w + 1 < _NWAVE)
            def _fire():
                fire_wave(w + 1, 1 - par)

            wait_wave(w, par)
        return 0

    lax.fori_loop(0, _NWAVE // 2, outer, 0)

    zero = jnp.zeros((_LANES,), jnp.float32)

    def rbody(j, carry):
        acc = list(carry)
        for g in range(_GPW):
            off = j * _BPW + g * _LANES
            acc[g] = acc[g] + g0_v[pl.ds(off, _LANES)]
            acc[_GPW + g] = acc[_GPW + g] + g1_v[pl.ds(off, _LANES)]
        return tuple(acc)

    acc = lax.fori_loop(0, _SEQ, rbody, (zero,) * (2 * _GPW))
    for g in range(_GPW):
        o0_v[pl.ds(g * _LANES, _LANES)] = acc[g] + b_v[0, :]
        o1_v[pl.ds(g * _LANES, _LANES)] = acc[_GPW + g] + b_v[1, :]

    pltpu.sync_copy(o0_v, out_hbm.at[pl.ds(base, _BPW)])
    pltpu.sync_copy(o1_v, out_hbm.at[pl.ds(_B + base, _BPW)])


@functools.partial(
    pl.kernel,
    out_type=jax.ShapeDtypeStruct((_C * _B,), jnp.float32),
    mesh=plsc.VectorSubcoreMesh(core_axis_name="c", subcore_axis_name="s"),
    scratch_types=[
        pltpu.VMEM((_SEQ, _BPW), jnp.int32),
        pltpu.VMEM((_SEQ * _BPW,), jnp.float32),
        pltpu.VMEM((_SEQ * _BPW,), jnp.float32),
        pltpu.VMEM((_C, _LANES), jnp.float32),
        pltpu.VMEM((_BPW,), jnp.float32),
        pltpu.VMEM((_BPW,), jnp.float32),
        pltpu.SemaphoreType.DMA,
        pltpu.SemaphoreType.DMA,
    ],
    compiler_params=pltpu.CompilerParams(use_tc_tiling_on_sc=False,
                                         needs_layout_passes=False),
)
def _sc_pool(xt_hbm, t0_hbm, t1_hbm, b_hbm, out_hbm,
             idx_v, g0_v, g1_v, b_v, o0_v, o1_v, sem0, sem1):
    _pool_body(xt_hbm, t0_hbm, t1_hbm, b_hbm, out_hbm,
               idx_v, g0_v, g1_v, b_v, o0_v, o1_v, sem0, sem1)


def kernel(x, table, W, b):
    w8 = jnp.zeros((8, _D), jnp.float32).at[:_C].set(
        W.astype(jnp.float32) * (1.0 / _SEQ))
    t0, t1 = _proj(w8, table.T)
    b_bcast = jnp.broadcast_to(b.astype(jnp.float32)[:, None], (_C, _LANES))
    out = _sc_pool(x.T.astype(jnp.int32), t0, t1, b_bcast)
    return out.reshape(_C, _B).T


# speedup vs baseline: 4.1963x; 1.2010x over previous
import functools

import jax
import jax.numpy as jnp
from jax import lax
from jax.experimental import pallas as pl
from jax.experimental.pallas import tpu as pltpu
from jax.experimental.pallas import tpu_sc as plsc

_VOCAB = 1000000
_D = 64
_C = 2
_B = 4096
_SEQ = 200

_NC = 2
_NS = 16
_NW = _NC * _NS
_BPW = _B // _NW
_LANES = 16
_GPW = _BPW // _LANES

_NB = 16384
_GRID = (_VOCAB + _NB - 1) // _NB


def _proj_body(w_ref, tt_ref, o0_ref, o1_ref):
    res = jnp.dot(w_ref[...], tt_ref[...], preferred_element_type=jnp.float32)
    o0_ref[...] = res[0, :]
    o1_ref[...] = res[1, :]


_proj = pl.pallas_call(
    _proj_body,
    grid=(_GRID,),
    in_specs=[
        pl.BlockSpec((8, _D), lambda i: (0, 0)),
        pl.BlockSpec((_D, _NB), lambda i: (0, i)),
    ],
    out_specs=[
        pl.BlockSpec((_NB,), lambda i: (i,)),
        pl.BlockSpec((_NB,), lambda i: (i,)),
    ],
    out_shape=[
        jax.ShapeDtypeStruct((_VOCAB,), jnp.float32),
        jax.ShapeDtypeStruct((_VOCAB,), jnp.float32),
    ],
)

_WAVE = 20
_NWAVE = _SEQ // _WAVE


def _pool_body(xt_hbm, t0_hbm, t1_hbm, b_hbm, out_hbm,
               idx_v, g0_v, g1_v, b_v, o0_v, o1_v, sem0, sem1):
    wid = lax.axis_index("s") * _NC + lax.axis_index("c")
    base = wid * _BPW

    pltpu.sync_copy(xt_hbm.at[:, pl.ds(base, _BPW)], idx_v)
    pltpu.sync_copy(b_hbm, b_v)

    sems = (sem0, sem1)
    planes = ((t0_hbm, g0_v), (t1_hbm, g1_v))

    def copies(j, sem):
        return [pltpu.make_async_copy(src.at[idx_v.at[j]],
                                      buf.at[pl.ds(j * _BPW, _BPW)], sem)
                for src, buf in planes]

    def fire_wave(w, par):
        def f1(j, _):
            for c in copies(j, sems[par]):
                c.start()
            return 0
        lax.fori_loop(w * _WAVE, (w + 1) * _WAVE, f1, 0)

    def wait_wave(w, par):
        def f1(j, _):
            for c in copies(j, sems[par]):
                c.wait()
            return 0
        lax.fori_loop(w * _WAVE, (w + 1) * _WAVE, f1, 0)

    fire_wave(0, 0)

    def outer(i, _):
        for par in range(2):
            w = 2 * i + par

            @pl.when(w + 1 < _NWAVE)
            def _fire():
                fire_wave(w + 1, 1 - par)

            wait_wave(w, par)
        return 0

    lax.fori_loop(0, _NWAVE // 2, outer, 0)

    zero = jnp.zeros((_LANES,), jnp.float32)

    def rbody(j, carry):
        acc = list(carry)
        for g in range(_GPW):
            off = j * _BPW + g * _LANES
            acc[g] = acc[g] + g0_v[pl.ds(off, _LANES)]
            acc[_GPW + g] = acc[_GPW + g] + g1_v[pl.ds(off, _LANES)]
        return tuple(acc)

    acc = lax.fori_loop(0, _SEQ, rbody, (zero,) * (2 * _GPW))
    for g in range(_GPW):
        o0_v[pl.ds(g * _LANES, _LANES)] = acc[g] + b_v[0, :]
        o1_v[pl.ds(g * _LANES, _LANES)] = acc[_GPW + g] + b_v[1, :]

    pltpu.sync_copy(o0_v, out_hbm.at[pl.ds(base, _BPW)])
    pltpu.sync_copy(o1_v, out_hbm.at[pl.ds(_B + base, _BPW)])


@functools.partial(
    pl.kernel,
    out_type=jax.ShapeDtypeStruct((_C * _B,), jnp.float32),
    mesh=plsc.VectorSubcoreMesh(core_axis_name="c", subcore_axis_name="s"),
    scratch_types=[
        pltpu.VMEM((_SEQ, _BPW), jnp.int32),
        pltpu.VMEM((_SEQ * _BPW,), jnp.float32),
        pltpu.VMEM((_SEQ * _BPW,), jnp.float32),
        pltpu.VMEM((_C, _LANES), jnp.float32),
        pltpu.VMEM((_BPW,), jnp.float32),
        pltpu.VMEM((_BPW,), jnp.float32),
        pltpu.SemaphoreType.DMA,
        pltpu.SemaphoreType.DMA,
    ],
    compiler_params=pltpu.CompilerParams(use_tc_tiling_on_sc=False,
                                         needs_layout_passes=False),
)
def _sc_pool(xt_hbm, t0_hbm, t1_hbm, b_hbm, out_hbm,
             idx_v, g0_v, g1_v, b_v, o0_v, o1_v, sem0, sem1):
    _pool_body(xt_hbm, t0_hbm, t1_hbm, b_hbm, out_hbm,
               idx_v, g0_v, g1_v, b_v, o0_v, o1_v, sem0, sem1)


def kernel(x, table, W, b):
    w8 = jnp.zeros((8, _D), jnp.float32).at[:_C].set(
        W.astype(jnp.float32) * (1.0 / _SEQ))
    t0, t1 = _proj(w8, table.T)
    b_bcast = jnp.broadcast_to(b.astype(jnp.float32)[:, None], (_C, _LANES))
    out = _sc_pool(x.T.astype(jnp.int32), t0, t1, b_bcast)
    return out.reshape(_C, _B).T


# speedup vs baseline: 4.4229x; 1.0540x over previous
import functools

import jax
import jax.numpy as jnp
from jax import lax
from jax.experimental import pallas as pl
from jax.experimental.pallas import tpu as pltpu
from jax.experimental.pallas import tpu_sc as plsc

_VOCAB = 1000000
_D = 64
_C = 2
_B = 4096
_SEQ = 200

_NC = 2
_NS = 16
_NW = _NC * _NS
_BPW = _B // _NW
_LANES = 16
_GPW = _BPW // _LANES

_NB = 32768
_GRID = (_VOCAB + _NB - 1) // _NB


def _proj_body(w_ref, tt_ref, o0_ref, o1_ref):
    res = jnp.dot(w_ref[...], tt_ref[...], preferred_element_type=jnp.float32)
    o0_ref[...] = res[0, :]
    o1_ref[...] = res[1, :]


_proj = pl.pallas_call(
    _proj_body,
    grid=(_GRID,),
    in_specs=[
        pl.BlockSpec((8, _D), lambda i: (0, 0)),
        pl.BlockSpec((_D, _NB), lambda i: (0, i)),
    ],
    out_specs=[
        pl.BlockSpec((_NB,), lambda i: (i,)),
        pl.BlockSpec((_NB,), lambda i: (i,)),
    ],
    out_shape=[
        jax.ShapeDtypeStruct((_VOCAB,), jnp.float32),
        jax.ShapeDtypeStruct((_VOCAB,), jnp.float32),
    ],
)

_WAVE = 20
_NWAVE = _SEQ // _WAVE


def _pool_body(xt_hbm, t0_hbm, t1_hbm, b_hbm, out_hbm,
               idx_v, g0_v, g1_v, b_v, o0_v, o1_v, sem0, sem1):
    wid = lax.axis_index("s") * _NC + lax.axis_index("c")
    base = wid * _BPW

    pltpu.sync_copy(xt_hbm.at[:, pl.ds(base, _BPW)], idx_v)
    pltpu.sync_copy(b_hbm, b_v)

    sems = (sem0, sem1)
    planes = ((t0_hbm, g0_v), (t1_hbm, g1_v))

    def copies(j, sem):
        return [pltpu.make_async_copy(src.at[idx_v.at[j]],
                                      buf.at[pl.ds(j * _BPW, _BPW)], sem)
                for src, buf in planes]

    def fire_wave(w, par):
        def f1(j, _):
            for c in copies(j, sems[par]):
                c.start()
            return 0
        lax.fori_loop(w * _WAVE, (w + 1) * _WAVE, f1, 0)

    def wait_wave(w, par):
        def f1(j, _):
            for c in copies(j, sems[par]):
                c.wait()
            return 0
        lax.fori_loop(w * _WAVE, (w + 1) * _WAVE, f1, 0)

    fire_wave(0, 0)

    def outer(i, _):
        for par in range(2):
            w = 2 * i + par

            @pl.when(w + 1 < _NWAVE)
            def _fire():
                fire_wave(w + 1, 1 - par)

            wait_wave(w, par)
        return 0

    lax.fori_loop(0, _NWAVE // 2, outer, 0)

    zero = jnp.zeros((_LANES,), jnp.float32)

    def rbody(j, carry):
        acc = list(carry)
        for g in range(_GPW):
            off = j * _BPW + g * _LANES
            acc[g] = acc[g] + g0_v[pl.ds(off, _LANES)]
            acc[_GPW + g] = acc[_GPW + g] + g1_v[pl.ds(off, _LANES)]
        return tuple(acc)

    acc = lax.fori_loop(0, _SEQ, rbody, (zero,) * (2 * _GPW))
    for g in range(_GPW):
        o0_v[pl.ds(g * _LANES, _LANES)] = acc[g] + b_v[0, :]
        o1_v[pl.ds(g * _LANES, _LANES)] = acc[_GPW + g] + b_v[1, :]

    pltpu.sync_copy(o0_v, out_hbm.at[pl.ds(base, _BPW)])
    pltpu.sync_copy(o1_v, out_hbm.at[pl.ds(_B + base, _BPW)])


@functools.partial(
    pl.kernel,
    out_type=jax.ShapeDtypeStruct((_C * _B,), jnp.float32),
    mesh=plsc.VectorSubcoreMesh(core_axis_name="c", subcore_axis_name="s"),
    scratch_types=[
        pltpu.VMEM((_SEQ, _BPW), jnp.int32),
        pltpu.VMEM((_SEQ * _BPW,), jnp.float32),
        pltpu.VMEM((_SEQ * _BPW,), jnp.float32),
        pltpu.VMEM((_C, _LANES), jnp.float32),
        pltpu.VMEM((_BPW,), jnp.float32),
        pltpu.VMEM((_BPW,), jnp.float32),
        pltpu.SemaphoreType.DMA,
        pltpu.SemaphoreType.DMA,
    ],
    compiler_params=pltpu.CompilerParams(use_tc_tiling_on_sc=False,
                                         needs_layout_passes=False),
)
def _sc_pool(xt_hbm, t0_hbm, t1_hbm, b_hbm, out_hbm,
             idx_v, g0_v, g1_v, b_v, o0_v, o1_v, sem0, sem1):
    _pool_body(xt_hbm, t0_hbm, t1_hbm, b_hbm, out_hbm,
               idx_v, g0_v, g1_v, b_v, o0_v, o1_v, sem0, sem1)


def kernel(x, table, W, b):
    w8 = jnp.zeros((8, _D), jnp.float32).at[:_C].set(
        W.astype(jnp.float32) * (1.0 / _SEQ))
    t0, t1 = _proj(w8, table.T)
    b_bcast = jnp.broadcast_to(b.astype(jnp.float32)[:, None], (_C, _LANES))
    out = _sc_pool(x.T.astype(jnp.int32), t0, t1, b_bcast)
    return out.reshape(_C, _B).T
